# Initial kernel scaffold; baseline (speedup 1.0000x reference)
import jax
import jax.numpy as jnp
from jax.experimental import pallas as pl


def kernel(idx, tok_table, pos_table):
    raise NotImplementedError("write your pallas kernel here")


# speedup vs baseline: 1.5268x; 1.5268x over previous
import functools

import jax
import jax.numpy as jnp
from jax import lax
from jax.experimental import pallas as pl
from jax.experimental.pallas import tpu as pltpu
from jax.experimental.pallas import tpu_sc as plsc


@functools.partial(jax.jit, static_argnums=(0, 1, 2))
def _embed(n, t, d_half, idx_flat, tok_table, pos_table):
    NW = 32
    assert n % NW == 0
    rpw = n // NW
    assert rpw % t == 0

    GSZ = 128
    CH = 640
    assert CH % GSZ == 0 and rpw % CH == 0
    NG = CH // GSZ
    NCH = rpw // CH

    PREP = 4
    PCH = PREP * t
    assert rpw % PCH == 0
    NPCH = rpw // PCH

    mesh = plsc.VectorSubcoreMesh(core_axis_name="c", subcore_axis_name="s")

    @functools.partial(
        pl.kernel,
        out_type=jax.ShapeDtypeStruct((n, 2 * d_half), jnp.float32),
        mesh=mesh,
        scratch_types=[
            pltpu.VMEM((CH,), jnp.int32),
            pltpu.VMEM((CH, d_half), jnp.float32),
            pltpu.VMEM((PCH, d_half), jnp.float32),
            pltpu.SemaphoreType.DMA,
        ],
        compiler_params=pltpu.CompilerParams(use_tc_tiling_on_sc=False),
    )
    def emb(idx_hbm, tok_hbm, pos_hbm, out_hbm, idx_v, tok_v, pos_v, sem):
        wid = lax.axis_index("s") * 2 + lax.axis_index("c")
        base = wid * rpw

        for r in range(PREP):
            pltpu.sync_copy(pos_hbm.at[pl.ds(0, t)], pos_v.at[pl.ds(r * t, t)])

        @pl.loop(0, NPCH)
        def pos_body(i):
            off = base + i * PCH
            pltpu.sync_copy(pos_v, out_hbm.at[pl.ds(off, PCH), pl.ds(d_half, d_half)])

        @pl.loop(0, NCH)
        def tok_body(i):
            off = base + i * CH
            pltpu.sync_copy(idx_hbm.at[pl.ds(off, CH)], idx_v)
            cps = [
                pltpu.async_copy(
                    tok_hbm.at[idx_v.at[pl.ds(j * GSZ, GSZ)]],
                    tok_v.at[pl.ds(j * GSZ, GSZ)],
                    sem,
                )
                for j in range(NG)
            ]
            for cp in cps:
                cp.wait()
            pltpu.sync_copy(tok_v, out_hbm.at[pl.ds(off, CH), pl.ds(0, d_half)])

    return emb(idx_flat, tok_table, pos_table)


def kernel(idx, tok_table, pos_table):
    B, T = idx.shape
    d_half = tok_table.shape[1]
    out = _embed(B * T, T, d_half, idx.reshape(-1).astype(jnp.int32),
                 tok_table, pos_table)
    return out.reshape(B, T, 2 * d_half)


# speedup vs baseline: 1.5910x; 1.0421x over previous
import functools

import jax
import jax.numpy as jnp
from jax import lax
from jax.experimental import pallas as pl
from jax.experimental.pallas import tpu as pltpu
from jax.experimental.pallas import tpu_sc as plsc


@functools.partial(jax.jit, static_argnums=(0, 1, 2))
def _embed(n, t, d_half, idx_flat, tok_table, pos_table):
    NW = 32
    assert n % NW == 0
    rpw = n // NW
    assert rpw % t == 0

    GSZ = 128
    CH = 640
    assert CH % GSZ == 0 and rpw % CH == 0 and (rpw // CH) % 2 == 0
    NG = CH // GSZ
    NCH = rpw // CH

    PREP = 4
    PCH = PREP * t
    assert rpw % PCH == 0
    NPCH = rpw // PCH
    assert NPCH <= NCH

    mesh = plsc.VectorSubcoreMesh(core_axis_name="c", subcore_axis_name="s")

    @functools.partial(
        pl.kernel,
        out_type=jax.ShapeDtypeStruct((n, 2 * d_half), jnp.float32),
        mesh=mesh,
        scratch_types=[
            pltpu.VMEM((2, CH), jnp.int32),
            pltpu.VMEM((2, CH, d_half), jnp.float32),
            pltpu.VMEM((PCH, d_half), jnp.float32),
            pltpu.SemaphoreType.DMA,
            pltpu.SemaphoreType.DMA,
            pltpu.SemaphoreType.DMA,
            pltpu.SemaphoreType.DMA,
            pltpu.SemaphoreType.DMA,
            pltpu.SemaphoreType.DMA,
            pltpu.SemaphoreType.DMA,
        ],
        compiler_params=pltpu.CompilerParams(use_tc_tiling_on_sc=False),
    )
    def emb(idx_hbm, tok_hbm, pos_hbm, out_hbm, idx_v, tok_v, pos_v,
            sem_i0, sem_i1, sem_g0, sem_g1, sem_w0, sem_w1, sem_p):
        sem_i = (sem_i0, sem_i1)
        sem_g = (sem_g0, sem_g1)
        sem_w = (sem_w0, sem_w1)
        wid = lax.axis_index("s") * 2 + lax.axis_index("c")
        base = wid * rpw

        for r in range(PREP):
            pltpu.sync_copy(pos_hbm.at[pl.ds(0, t)], pos_v.at[pl.ds(r * t, t)])

        pltpu.async_copy(idx_hbm.at[pl.ds(base, CH)], idx_v.at[0], sem_i[0])

        def sub(gg, b):
            off = base + gg * CH

            @pl.when(gg >= 2)
            def _():
                pltpu.make_async_copy(
                    tok_v.at[b],
                    out_hbm.at[pl.ds(off, CH), pl.ds(0, d_half)],
                    sem_w[b],
                ).wait()

            pltpu.make_async_copy(
                idx_hbm.at[pl.ds(off, CH)], idx_v.at[b], sem_i[b]
            ).wait()
            for j in range(NG):
                pltpu.async_copy(
                    tok_hbm.at[idx_v.at[b, pl.ds(j * GSZ, GSZ)]],
                    tok_v.at[b, pl.ds(j * GSZ, GSZ)],
                    sem_g[b],
                )

            @pl.when(gg + 1 < NCH)
            def _():
                pltpu.async_copy(
                    idx_hbm.at[pl.ds(off + CH, CH)], idx_v.at[1 - b],
                    sem_i[1 - b],
                )

            @pl.when(gg < NPCH)
            def _():
                pltpu.async_copy(
                    pos_v,
                    out_hbm.at[pl.ds(base + gg * PCH, PCH),
                               pl.ds(d_half, d_half)],
                    sem_p,
                )

            for j in range(NG):
                pltpu.make_async_copy(
                    tok_hbm.at[idx_v.at[b, pl.ds(j * GSZ, GSZ)]],
                    tok_v.at[b, pl.ds(j * GSZ, GSZ)],
                    sem_g[b],
                ).wait()
            pltpu.async_copy(
                tok_v.at[b],
                out_hbm.at[pl.ds(off, CH), pl.ds(0, d_half)],
                sem_w[b],
            )

        @pl.loop(0, NCH, step=2)
        def body(g):
            sub(g, 0)
            sub(g + 1, 1)

        for b in range(2):
            pltpu.make_async_copy(
                tok_v.at[b],
                out_hbm.at[pl.ds(base, CH), pl.ds(0, d_half)],
                sem_w[b],
            ).wait()

        @pl.loop(0, NPCH)
        def drain_pos(i):
            pltpu.make_async_copy(
                pos_v,
                out_hbm.at[pl.ds(base, PCH), pl.ds(d_half, d_half)],
                sem_p,
            ).wait()

    return emb(idx_flat, tok_table, pos_table)


def kernel(idx, tok_table, pos_table):
    B, T = idx.shape
    d_half = tok_table.shape[1]
    out = _embed(B * T, T, d_half, idx.reshape(-1).astype(jnp.int32),
                 tok_table, pos_table)
    return out.reshape(B, T, 2 * d_half)
